# baseline (speedup 1.0000x reference)
import functools

import jax
import jax.numpy as jnp
from jax import lax
from jax.experimental import pallas as pl
from jax.experimental.pallas import tpu as pltpu
from jax.experimental.pallas import tpu_sc as plsc

_NC = 2
_NS = 16


def _sc_bcast(table, Bsc):
    S, D = table.shape
    NW = _NC * _NS
    rows_per_w = S // NW
    CH = 32
    n_ch = rows_per_w // CH
    mesh = plsc.VectorSubcoreMesh(core_axis_name="c", subcore_axis_name="s")

    @functools.partial(
        pl.kernel,
        mesh=mesh,
        out_type=jax.ShapeDtypeStruct((Bsc, S, D), jnp.float32),
        scratch_types=[
            pltpu.VMEM((CH, D), jnp.float32),
            pltpu.SemaphoreType.DMA,
        ],
    )
    def sc_body(table_hbm, out_hbm, buf, sem):
        wid = lax.axis_index("s") * _NC + lax.axis_index("c")
        base = wid * rows_per_w

        def body(i, carry):
            r0 = base + i * CH
            pltpu.sync_copy(table_hbm.at[pl.ds(r0, CH)], buf)
            for b in range(Bsc):
                pltpu.sync_copy(buf, out_hbm.at[b, pl.ds(r0, CH)])
            return carry

        lax.fori_loop(0, n_ch, body, 0)

    return sc_body(table)


def _tc_body(table_ref, out_ref):
    out_ref[...] = jnp.broadcast_to(table_ref[...][None, :, :], out_ref.shape)


def _tc_bcast(table, Btc):
    S, D = table.shape
    BLK_S = 1024
    return pl.pallas_call(
        _tc_body,
        grid=(S // BLK_S,),
        in_specs=[pl.BlockSpec((BLK_S, D), lambda j: (j, 0))],
        out_specs=pl.BlockSpec((Btc, BLK_S, D), lambda j: (0, j, 0)),
        out_shape=jax.ShapeDtypeStruct((Btc, S, D), table.dtype),
    )(table)


def kernel(hidden_states, position_embeddings):
    B, S, D = hidden_states.shape
    assert position_embeddings.shape == (S, D)
    Bsc = B // 2
    sc_out = _sc_bcast(position_embeddings, Bsc)
    tc_out = _tc_bcast(position_embeddings, B - Bsc)
    return jnp.concatenate([sc_out, tc_out], axis=0)


# speedup vs baseline: 2.9734x; 2.9734x over previous
import jax
import jax.numpy as jnp
from jax.experimental import pallas as pl


def _copy(table_ref, out_ref):
    out_ref[...] = table_ref[...][None, :, :]


def kernel(hidden_states, position_embeddings):
    B, S, D = hidden_states.shape
    assert position_embeddings.shape == (S, D)
    BLK_S = 2048
    return pl.pallas_call(
        _copy,
        grid=(S // BLK_S, B),
        in_specs=[pl.BlockSpec((BLK_S, D), lambda j, i: (j, 0))],
        out_specs=pl.BlockSpec((1, BLK_S, D), lambda j, i: (i, j, 0)),
        out_shape=jax.ShapeDtypeStruct((B, S, D), position_embeddings.dtype),
    )(position_embeddings)
